# baseline (speedup 1.0000x reference)
import jax
import jax.numpy as jnp
from jax.experimental import pallas as pl
from jax.experimental.pallas import tpu as pltpu

_NUM_VECTOR = 8
_LAYER_IDX = 16
_TAIL = 8
_CHUNKS = (1360, 1360, 1368)


def _dma_route_kernel(x_ref, w_ref, b_ref, vp_ref, out_ref,
                      tail, bulk_sem, row_sem, *, batch, seq):
    bulk = []
    k = 0
    for b in range(batch):
        off = b * seq
        for sz in _CHUNKS:
            cp = pltpu.make_async_copy(
                x_ref.at[pl.ds(off, sz), :],
                out_ref.at[pl.ds(off, sz), :],
                bulk_sem.at[k])
            cp.start()
            bulk.append(cp)
            off += sz
            k += 1
    gets = []
    for b in range(batch):
        cp = pltpu.make_async_copy(
            x_ref.at[pl.ds(b * seq + seq - _TAIL, _TAIL), :],
            tail.at[b],
            row_sem.at[b])
        cp.start()
        gets.append(cp)
    for cp in gets:
        cp.wait()
    token_act = tail[:, _TAIL - 1, :]
    scores = jnp.dot(token_act, w_ref[...],
                     preferred_element_type=jnp.float32) + b_ref[...]
    probs = jax.nn.softmax(scores, axis=-1)
    idx = jnp.argmax(probs, axis=-1)
    keep = (idx == _NUM_VECTOR)[:, None]
    onehot = (jax.lax.broadcasted_iota(jnp.int32, (batch, _NUM_VECTOR), 1)
              == jnp.minimum(idx, _NUM_VECTOR - 1)[:, None]).astype(jnp.float32)
    repl = jnp.dot(onehot, vp_ref[...], preferred_element_type=jnp.float32)
    tail[:, _TAIL - 1, :] = jnp.where(keep, token_act, repl)
    puts = []
    for b in range(batch):
        cp = pltpu.make_async_copy(
            tail.at[b],
            out_ref.at[pl.ds(b * seq + seq - _TAIL, _TAIL), :],
            row_sem.at[b])
        cp.start()
        puts.append(cp)
    for cp in puts:
        cp.wait()
    for cp in bulk:
        cp.wait()


def kernel(x, vector_pool, gate_W, gate_b):
    B, S, H = x.shape
    vp16 = vector_pool[:, _LAYER_IDX, :]
    gate_b2 = gate_b.reshape(1, -1)
    x2 = x.reshape(B * S, H)
    nbulk = B * len(_CHUNKS)

    def body(x_ref, w_ref, b_ref, vp_ref, out_ref, tail, bulk_sem, row_sem):
        _dma_route_kernel(x_ref, w_ref, b_ref, vp_ref, out_ref,
                          tail, bulk_sem, row_sem, batch=B, seq=S)

    out2 = pl.pallas_call(
        body,
        in_specs=[
            pl.BlockSpec(memory_space=pltpu.HBM),
            pl.BlockSpec(memory_space=pltpu.VMEM),
            pl.BlockSpec(memory_space=pltpu.VMEM),
            pl.BlockSpec(memory_space=pltpu.VMEM),
        ],
        out_specs=pl.BlockSpec(memory_space=pltpu.HBM),
        out_shape=jax.ShapeDtypeStruct((B * S, H), x.dtype),
        scratch_shapes=[
            pltpu.VMEM((B, _TAIL, H), jnp.float32),
            pltpu.SemaphoreType.DMA((nbulk,)),
            pltpu.SemaphoreType.DMA((B,)),
        ],
    )(x2, gate_W, gate_b2, vp16)
    return out2.reshape(B, S, H)


# speedup vs baseline: 46.5047x; 46.5047x over previous
import functools

import jax
import jax.numpy as jnp
from jax.experimental import pallas as pl

_NUM_VECTOR = 8
_LAYER_IDX = 16
_ROWS = 1024


def _copy_route_kernel(x_ref, w_ref, b_ref, vp_ref, out_ref, *, per_batch):
    j = pl.program_id(0)
    out_ref[...] = x_ref[...]

    @pl.when(j % per_batch == per_batch - 1)
    def _route():
        token_act = x_ref[_ROWS - 1, :].reshape(1, -1)
        scores = jnp.dot(token_act, w_ref[...],
                         preferred_element_type=jnp.float32) + b_ref[...]
        probs = jax.nn.softmax(scores, axis=-1)
        idx = jnp.argmax(probs[0, :])
        keep = idx == _NUM_VECTOR
        onehot = (jax.lax.broadcasted_iota(jnp.int32, (1, _NUM_VECTOR), 1)
                  == jnp.minimum(idx, _NUM_VECTOR - 1)).astype(jnp.float32)
        repl = jnp.dot(onehot, vp_ref[...],
                       preferred_element_type=jnp.float32)
        out_ref[_ROWS - 1, :] = jnp.where(keep, token_act, repl)[0]


def kernel(x, vector_pool, gate_W, gate_b):
    B, S, H = x.shape
    vp16 = vector_pool[:, _LAYER_IDX, :]
    gate_b2 = gate_b.reshape(1, -1)
    x2 = x.reshape(B * S, H)
    nblk = (B * S) // _ROWS
    per_batch = S // _ROWS
    out2 = pl.pallas_call(
        functools.partial(_copy_route_kernel, per_batch=per_batch),
        grid=(nblk,),
        in_specs=[
            pl.BlockSpec((_ROWS, H), lambda j: (j, 0)),
            pl.BlockSpec((H, _NUM_VECTOR + 1), lambda j: (0, 0)),
            pl.BlockSpec((1, _NUM_VECTOR + 1), lambda j: (0, 0)),
            pl.BlockSpec((_NUM_VECTOR, H), lambda j: (0, 0)),
        ],
        out_specs=pl.BlockSpec((_ROWS, H), lambda j: (j, 0)),
        out_shape=jax.ShapeDtypeStruct((B * S, H), x.dtype),
    )(x2, gate_W, gate_b2, vp16)
    return out2.reshape(B, S, H)
